# baseline (speedup 1.0000x reference)
import functools
import math

import jax
import jax.numpy as jnp
from jax.experimental import pallas as pl
from jax.experimental.pallas import tpu as pltpu
from jax.experimental.pallas import tpu_sc as plsc

_N1, _N2, _N3 = 2060, 2459, 3929
_D, _R, _EMB, _H = 901, 1778, 3604, 4
_DP = 1024
_SP = 1792
_EMBP = _H * _DP
_CP = 3840
_EPS = 1e-5


def _pad2(x, r, c):
    return jnp.pad(x, ((0, r - x.shape[0]), (0, c - x.shape[1])))


def _rowvec(v, n):
    v = jnp.pad(v, (0, n - v.shape[0]))
    return jnp.broadcast_to(v[None, :], (8, n)).astype(jnp.float32)


def _dot(a, b):
    return jax.lax.dot_general(a, b, (((1,), (0,)), ((), ())),
                               preferred_element_type=jnp.float32)


def _dot_t(a, b):
    return jax.lax.dot_general(a, b, (((1,), (1,)), ((), ())),
                               preferred_element_type=jnp.float32)


def _h_body(n_real, bm, x_ref, w_ref, av_ref, h_ref, al_ref):
    x = x_ref[...]
    lane = jax.lax.broadcasted_iota(jnp.int32, x.shape, 1)
    x = jnp.where(lane < n_real, x, 0.0).astype(jnp.bfloat16)
    h = _dot(x, w_ref[...])
    grow = (jax.lax.broadcasted_iota(jnp.int32, h.shape, 0)
            + pl.program_id(0) * bm)
    hb = jnp.where(grow < n_real, h, 0.0).astype(jnp.bfloat16)
    h_ref[...] = hb
    al_ref[...] = _dot(hb, av_ref[...])


def _gat_h(x, w_p, av, n_real, np_, bm):
    dp = w_p.shape[1]
    return pl.pallas_call(
        functools.partial(_h_body, n_real, bm),
        grid=(np_ // bm,),
        in_specs=[
            pl.BlockSpec((bm, np_), lambda i: (i, 0)),
            pl.BlockSpec((np_, dp), lambda i: (0, 0)),
            pl.BlockSpec((dp, 128), lambda i: (0, 0)),
        ],
        out_specs=[
            pl.BlockSpec((bm, dp), lambda i: (i, 0)),
            pl.BlockSpec((bm, 128), lambda i: (i, 0)),
        ],
        out_shape=[
            jax.ShapeDtypeStruct((np_, dp), jnp.bfloat16),
            jax.ShapeDtypeStruct((np_, 128), jnp.float32),
        ],
    )(x, w_p, av)


def _gat_attn_body(n_real, adj_ref, alst_ref, al_ref, h_ref, b_ref, o_ref):
    adjt = jnp.transpose(adj_ref[...])
    ad = al_ref[:, 1:2]
    als = alst_ref[0:1, :]
    e = ad + als
    e = jnp.where(e >= 0, e, 0.2 * e)
    e = jnp.where(adjt != 0, e, -1e9)
    col = jax.lax.broadcasted_iota(jnp.int32, e.shape, 1)
    e = jnp.where(col < n_real, e, -2e9)
    m = jnp.max(e, axis=1, keepdims=True)
    p = jnp.exp(e - m)
    s = jnp.sum(p, axis=1, keepdims=True)
    o = _dot((p / s).astype(jnp.bfloat16), h_ref[...])
    o_ref[...] = jnp.maximum(o + b_ref[0:1, :], 0.0).astype(jnp.bfloat16)


def _gat_attn(adj, alst, al, h_p, brow, n_real, np_, bm):
    dp = h_p.shape[1]
    return pl.pallas_call(
        functools.partial(_gat_attn_body, n_real),
        grid=(np_ // bm,),
        in_specs=[
            pl.BlockSpec((np_, bm), lambda i: (0, i)),
            pl.BlockSpec((8, np_), lambda i: (0, 0)),
            pl.BlockSpec((bm, 128), lambda i: (i, 0)),
            pl.BlockSpec((np_, dp), lambda i: (0, 0)),
            pl.BlockSpec((8, dp), lambda i: (0, 0)),
        ],
        out_specs=pl.BlockSpec((bm, dp), lambda i: (i, 0)),
        out_shape=jax.ShapeDtypeStruct((np_, dp), jnp.bfloat16),
    )(adj, alst, al, h_p, brow)


def _gat(x, adj, w, a_src, a_dst, b, n, np_):
    w_p = _pad2(w, np_, _DP).astype(jnp.bfloat16)
    av = jnp.zeros((_DP, 128), jnp.float32)
    av = av.at[: _D, 0].set(a_src).at[: _D, 1].set(a_dst)
    av = av.astype(jnp.bfloat16)
    h_p, al = _gat_h(x, w_p, av, n, np_, 256)
    alst = jnp.broadcast_to(al[:, 0][None, :], (8, np_))
    brow = _rowvec(b, _DP)
    return _gat_attn(adj, alst, al, h_p, brow, n, np_, 256)


def _qkv_body(x_ref, wqt_ref, wkt_ref, wvt_ref, q_ref, k_ref, v_ref):
    x = x_ref[0]
    q_ref[0] = _dot_t(x, wqt_ref[...]).astype(jnp.bfloat16)
    k_ref[0] = _dot_t(x, wkt_ref[...]).astype(jnp.bfloat16)
    v_ref[0] = _dot_t(x, wvt_ref[...]).astype(jnp.bfloat16)


def _qkv(sum_x_p, wqt, wkt, wvt, bm):
    grid = (_H, _SP // bm)
    io = pl.BlockSpec((1, bm, _DP), lambda h, r: (h, r, 0))
    return pl.pallas_call(
        _qkv_body,
        grid=grid,
        in_specs=[
            pl.BlockSpec((1, bm, _DP), lambda h, r: (h, r, 0)),
            pl.BlockSpec((_DP, _DP), lambda h, r: (0, 0)),
            pl.BlockSpec((_DP, _DP), lambda h, r: (0, 0)),
            pl.BlockSpec((_DP, _DP), lambda h, r: (0, 0)),
        ],
        out_specs=[io, io, io],
        out_shape=[jax.ShapeDtypeStruct((_H, _SP, _DP), jnp.bfloat16)] * 3,
    )(sum_x_p, wqt, wkt, wvt)


def _mha_body(q_ref, k_ref, v_ref, o_ref):
    q = q_ref[0]
    logits = _dot_t(q, k_ref[0]) * (1.0 / math.sqrt(_D))
    col = jax.lax.broadcasted_iota(jnp.int32, logits.shape, 1)
    logits = jnp.where(col < _R, logits, -1e30)
    m = jnp.max(logits, axis=1, keepdims=True)
    p = jnp.exp(logits - m)
    p = p / jnp.sum(p, axis=1, keepdims=True)
    o_ref[0] = _dot(p.astype(jnp.bfloat16), v_ref[0]).astype(jnp.bfloat16)


def _mha(q, k, v, bm):
    grid = (_H, _SP // bm)
    return pl.pallas_call(
        _mha_body,
        grid=grid,
        in_specs=[
            pl.BlockSpec((1, bm, _DP), lambda h, r: (h, r, 0)),
            pl.BlockSpec((1, _SP, _DP), lambda h, r: (h, 0, 0)),
            pl.BlockSpec((1, _SP, _DP), lambda h, r: (h, 0, 0)),
        ],
        out_specs=pl.BlockSpec((1, bm, _DP), lambda h, r: (h, r, 0)),
        out_shape=jax.ShapeDtypeStruct((_H, _SP, _DP), jnp.bfloat16),
    )(q, k, v)


def _wo_body(a_ref, w_ref, b_ref, o_ref, acc_ref):
    hh = pl.program_id(1)

    @pl.when(hh == 0)
    def _():
        acc_ref[...] = jnp.zeros_like(acc_ref)

    acc_ref[...] += _dot_t(a_ref[0], w_ref[0])

    @pl.when(hh == _H - 1)
    def _():
        o_ref[...] = acc_ref[...] + b_ref[0:1, :]


def _wo_proj(attn_h, wo_h, brow, bn):
    return pl.pallas_call(
        _wo_body,
        grid=(_CP // bn, _H),
        in_specs=[
            pl.BlockSpec((1, _SP, _DP), lambda j, hh: (hh, 0, 0)),
            pl.BlockSpec((1, bn, _DP), lambda j, hh: (hh, j, 0)),
            pl.BlockSpec((8, bn), lambda j, hh: (0, j)),
        ],
        out_specs=pl.BlockSpec((_SP, bn), lambda j, hh: (0, j)),
        out_shape=jax.ShapeDtypeStruct((_SP, _CP), jnp.float32),
        scratch_shapes=[pltpu.VMEM((_SP, bn), jnp.float32)],
    )(attn_h, wo_h, brow)


def _mm_body(nk, act, a_ref, wt_ref, b_ref, o_ref, acc_ref):
    @pl.when(pl.program_id(1) == 0)
    def _():
        acc_ref[...] = jnp.zeros_like(acc_ref)

    acc_ref[...] += _dot_t(a_ref[...], wt_ref[...])

    @pl.when(pl.program_id(1) == nk - 1)
    def _():
        o_ref[...] = act(acc_ref[...] + b_ref[0:1, :]).astype(o_ref.dtype)


def _mm_bias(a, wt, brow, act, bn, bk, out_dtype=jnp.float32):
    m, k = a.shape
    n = wt.shape[0]
    nk = k // bk
    return pl.pallas_call(
        functools.partial(_mm_body, nk, act),
        grid=(n // bn, nk),
        in_specs=[
            pl.BlockSpec((m, bk), lambda j, kk: (0, kk)),
            pl.BlockSpec((bn, bk), lambda j, kk: (j, kk)),
            pl.BlockSpec((8, bn), lambda j, kk: (0, j)),
        ],
        out_specs=pl.BlockSpec((m, bn), lambda j, kk: (0, j)),
        out_shape=jax.ShapeDtypeStruct((m, n), out_dtype),
        scratch_shapes=[pltpu.VMEM((m, bn), jnp.float32)],
    )(a, wt, brow)


def _sc_gather(x_p, idx):
    nidx = idx.shape[0]
    cols = x_p.shape[1]
    orig_dtype = x_p.dtype
    if x_p.dtype == jnp.bfloat16:
        x_p = jax.lax.bitcast_convert_type(
            x_p.reshape(x_p.shape[0], cols // 2, 2), jnp.int32)
        cols = cols // 2
    rec = 256
    rpr = cols // rec
    gw = 128
    nrec = nidx * rpr
    x_r = x_p.reshape(-1, rec)
    idx_r = (idx[:, None] * rpr
             + jnp.arange(rpr, dtype=jnp.int32)[None, :]).reshape(1, nrec)
    mesh = plsc.VectorSubcoreMesh(core_axis_name="c", subcore_axis_name="s")

    @functools.partial(
        pl.kernel,
        out_type=jax.ShapeDtypeStruct((nrec, rec), x_p.dtype),
        mesh=mesh,
    )
    def k(x_hbm, i_hbm, o_hbm):
        def body(i_vmem, o_vmem):
            pltpu.sync_copy(x_hbm.at[i_vmem.at[0]], o_vmem)

        pltpu.emit_pipeline(
            body,
            grid=(nrec // gw,),
            in_specs=[pl.BlockSpec((1, gw), index_map=lambda i: (0, i))],
            out_specs=[pl.BlockSpec((gw, rec), index_map=lambda i: (i, 0))],
            core_axis_name=("c", "s"),
            dimension_semantics=(pltpu.PARALLEL,),
        )(i_hbm, o_hbm)

    out = k(x_r, idx_r).reshape(nidx, cols)
    if orig_dtype == jnp.bfloat16:
        out = jax.lax.bitcast_convert_type(
            out, jnp.bfloat16).reshape(nidx, cols * 2)
    return out


def _stats_body(bsz, mirror, g_ref, b_ref, x_ref, alpha_ref, beta_ref,
                xb_ref=None):
    x = x_ref[...].astype(jnp.float32)
    if mirror:
        xb_ref[...] = x.astype(jnp.bfloat16)
    s = jnp.sum(x, axis=0, keepdims=True)
    ss = jnp.sum(x * x, axis=0, keepdims=True)
    mu = s * (1.0 / bsz)
    var = ss * (1.0 / bsz) - mu * mu
    al = g_ref[0:1, :] * jax.lax.rsqrt(var + _EPS)
    be = b_ref[0:1, :] - mu * al
    alpha_ref[...] = jnp.broadcast_to(al, alpha_ref.shape)
    beta_ref[...] = jnp.broadcast_to(be, beta_ref.shape)


def _bn_stats(x, grow, brow, bn, mirror=True):
    bsz, c = x.shape
    outs = [
        pl.BlockSpec((8, bn), lambda j: (0, j)),
        pl.BlockSpec((8, bn), lambda j: (0, j)),
    ]
    shapes = [jax.ShapeDtypeStruct((8, c), jnp.float32)] * 2
    if mirror:
        outs.append(pl.BlockSpec((bsz, bn), lambda j: (0, j)))
        shapes.append(jax.ShapeDtypeStruct((bsz, c), jnp.bfloat16))
    return pl.pallas_call(
        functools.partial(_stats_body, float(bsz), mirror),
        grid=(c // bn,),
        in_specs=[
            pl.BlockSpec((8, bn), lambda j: (0, j)),
            pl.BlockSpec((8, bn), lambda j: (0, j)),
            pl.BlockSpec((bsz, bn), lambda j: (0, j)),
        ],
        out_specs=outs,
        out_shape=shapes,
    )(grow, brow, x)


def _lin1_body(nk, a0_ref, a1_ref, al0_ref, be0_ref, al1_ref, be1_ref,
               w0_ref, w1_ref, bl_ref, o_ref, acc_ref):
    @pl.when(pl.program_id(2) == 0)
    def _():
        acc_ref[...] = jnp.zeros_like(acc_ref)

    z0 = a0_ref[...].astype(jnp.float32) * al0_ref[0:1, :] + be0_ref[0:1, :]
    z1 = a1_ref[...].astype(jnp.float32) * al1_ref[0:1, :] + be1_ref[0:1, :]
    acc_ref[...] += (_dot_t(z0.astype(jnp.bfloat16), w0_ref[...])
                     + _dot_t(z1.astype(jnp.bfloat16), w1_ref[...]))

    @pl.when(pl.program_id(2) == nk - 1)
    def _():
        r = acc_ref[...] + bl_ref[0:1, :]
        o_ref[...] = jnp.where(r >= 0, r, 0.01 * r).astype(jnp.bfloat16)


def _lin1(a0, a1, al0, be0, al1, be1, w0t, w1t, blrow, bm, bn, bk):
    bsz = a0.shape[0]
    kdim = a0.shape[1]
    n = w0t.shape[0]
    nk = kdim // bk
    return pl.pallas_call(
        functools.partial(_lin1_body, nk),
        grid=(bsz // bm, n // bn, nk),
        in_specs=[
            pl.BlockSpec((bm, bk), lambda i, j, kk: (i, kk)),
            pl.BlockSpec((bm, bk), lambda i, j, kk: (i, kk)),
            pl.BlockSpec((8, bk), lambda i, j, kk: (0, kk)),
            pl.BlockSpec((8, bk), lambda i, j, kk: (0, kk)),
            pl.BlockSpec((8, bk), lambda i, j, kk: (0, kk)),
            pl.BlockSpec((8, bk), lambda i, j, kk: (0, kk)),
            pl.BlockSpec((bn, bk), lambda i, j, kk: (j, kk)),
            pl.BlockSpec((bn, bk), lambda i, j, kk: (j, kk)),
            pl.BlockSpec((8, bn), lambda i, j, kk: (0, j)),
        ],
        out_specs=pl.BlockSpec((bm, bn), lambda i, j, kk: (i, j)),
        out_shape=jax.ShapeDtypeStruct((bsz, n), jnp.bfloat16),
        scratch_shapes=[pltpu.VMEM((bm, bn), jnp.float32)],
    )(a0, a1, al0, be0, al1, be1, w0t, w1t, blrow)


def _fin_body(y_ref, al_ref, be_ref, w2_ref, bl2_ref, o_ref):
    z = y_ref[...].astype(jnp.float32) * al_ref[0:1, :] + be_ref[0:1, :]
    r = _dot(z.astype(jnp.bfloat16), w2_ref[...]) + bl2_ref[0:1, :]
    o_ref[...] = jax.nn.sigmoid(r)


def _fin(y, al2, be2, w2col, bl2row, bm):
    bsz, c = y.shape
    return pl.pallas_call(
        _fin_body,
        grid=(bsz // bm,),
        in_specs=[
            pl.BlockSpec((bm, c), lambda i: (i, 0)),
            pl.BlockSpec((8, c), lambda i: (0, 0)),
            pl.BlockSpec((8, c), lambda i: (0, 0)),
            pl.BlockSpec((c, 128), lambda i: (0, 0)),
            pl.BlockSpec((8, 128), lambda i: (0, 0)),
        ],
        out_specs=pl.BlockSpec((bm, 128), lambda i: (i, 0)),
        out_shape=jax.ShapeDtypeStruct((bsz, 128), jnp.float32),
    )(y, al2, be2, w2col, bl2row)


def _mlp_scores(x0, x1, g1, b1, w0t, w1t, blrow, g2row, b2row, w2col,
                bl2row, bm):
    al0, be0, x0b = _bn_stats(x0, g1[0], b1[0], 768)
    al1, be1, x1b = _bn_stats(x1, g1[1], b1[1], 768)
    y = _lin1(x0b, x1b, al0, be0, al1, be1, w0t, w1t, blrow, bm, 1024, 1280)
    al2, be2 = _bn_stats(y, g2row, b2row, 512, mirror=False)
    return _fin(y, al2, be2, w2col, bl2row, 256)


def kernel(m_drug_d_adj, m_incRNA_d_adj, m_mRNA_d_adj, miRNA_disease_feature,
           x_drug, x_inc, x_mrna,
           Wd, a_src_d, a_dst_d, bd,
           Wi, a_src_i, a_dst_i, bi,
           Wm, a_src_m, a_dst_m, bm_,
           Wq, Wk, Wv, Wo, bo,
           g1, b1, W1, bl1, g2, b2, W2, bl2,
           train_sample, test_sample):
    f32 = jnp.float32
    hd = _gat(x_drug.astype(f32), m_drug_d_adj, Wd, a_src_d, a_dst_d, bd,
              _N1, 2304)
    hi = _gat(x_inc.astype(f32), m_incRNA_d_adj, Wi, a_src_i, a_dst_i, bi,
              _N2, 2560)
    hm = _gat(x_mrna.astype(f32), m_mRNA_d_adj, Wm, a_src_m, a_dst_m, bm_,
              _N3, 4096)

    d_s = jnp.concatenate([hd[:901], hd[1183:_N1]], axis=0)
    i_s = jnp.concatenate([hi[:901], hi[1582:_N2]], axis=0)
    m_s = jnp.concatenate([hm[:901], hm[3052:_N3]], axis=0)
    feat = _pad2(miRNA_disease_feature, _R, _DP).astype(jnp.bfloat16)
    x_heads = jnp.stack([
        jnp.pad(d_s, ((0, _SP - _R), (0, 0))),
        jnp.pad(i_s, ((0, _SP - _R), (0, 0))),
        jnp.pad(m_s, ((0, _SP - _R), (0, 0))),
        jnp.pad(feat, ((0, _SP - _R), (0, 0))),
    ])

    wqt = _pad2(Wq, _DP, _DP).astype(jnp.bfloat16)
    wkt = _pad2(Wk, _DP, _DP).astype(jnp.bfloat16)
    wvt = _pad2(Wv, _DP, _DP).astype(jnp.bfloat16)
    q, k, v = _qkv(x_heads, wqt, wkt, wvt, 256)
    attn = _mha(q, k, v, 256)

    wo_h = jnp.stack([
        _pad2(Wo[:, hh * _D:(hh + 1) * _D], _CP, _DP) for hh in range(_H)
    ]).astype(jnp.bfloat16)
    sum_x = _wo_proj(attn, wo_h, _rowvec(bo, _CP), 768)

    ts = train_sample.astype(jnp.int32)
    us = test_sample.astype(jnp.int32)
    tr0 = _sc_gather(sum_x, ts[:, 0])
    tr1 = _sc_gather(sum_x, ts[:, 1])
    te0 = _sc_gather(sum_x, us[:, 0])
    te1 = _sc_gather(sum_x, us[:, 1])

    g1h = (_rowvec(g1[:_EMB], _CP), _rowvec(g1[_EMB:], _CP))
    b1h = (_rowvec(b1[:_EMB], _CP), _rowvec(b1[_EMB:], _CP))
    w0t = _pad2(W1[:, :_EMB], _EMBP, _CP).astype(jnp.bfloat16)
    w1t = _pad2(W1[:, _EMB:], _EMBP, _CP).astype(jnp.bfloat16)
    blrow = _rowvec(bl1, _EMBP)
    g2row = _rowvec(g2, _EMBP)
    b2row = _rowvec(b2, _EMBP)
    w2col = jnp.zeros((_EMBP, 128), f32).at[:_EMB, 0].set(W2[0]).astype(jnp.bfloat16)
    bl2row = jnp.broadcast_to(bl2[0], (8, 128)).astype(f32)

    tr_score = _mlp_scores(tr0, tr1, g1h, b1h, w0t, w1t, blrow,
                           g2row, b2row, w2col, bl2row, 1024)
    te_score = _mlp_scores(te0, te1, g1h, b1h, w0t, w1t, blrow,
                           g2row, b2row, w2col, bl2row, 1024)
    return tr_score[:, :1], te_score[:, :1]


# speedup vs baseline: 1.0300x; 1.0300x over previous
import functools
import math

import jax
import jax.numpy as jnp
from jax.experimental import pallas as pl
from jax.experimental.pallas import tpu as pltpu
from jax.experimental.pallas import tpu_sc as plsc

_N1, _N2, _N3 = 2060, 2459, 3929
_D, _R, _EMB, _H = 901, 1778, 3604, 4
_DP = 1024
_SP = 1792
_EMBP = _H * _DP
_CP = 3840
_EPS = 1e-5


def _pad2(x, r, c):
    return jnp.pad(x, ((0, r - x.shape[0]), (0, c - x.shape[1])))


def _rowvec(v, n):
    v = jnp.pad(v, (0, n - v.shape[0]))
    return jnp.broadcast_to(v[None, :], (8, n)).astype(jnp.float32)


def _dot(a, b):
    return jax.lax.dot_general(a, b, (((1,), (0,)), ((), ())),
                               preferred_element_type=jnp.float32)


def _dot_t(a, b):
    return jax.lax.dot_general(a, b, (((1,), (1,)), ((), ())),
                               preferred_element_type=jnp.float32)


def _h_body(n_real, bm, x_ref, w_ref, av_ref, h_ref, al_ref):
    x = x_ref[...]
    lane = jax.lax.broadcasted_iota(jnp.int32, x.shape, 1)
    x = jnp.where(lane < n_real, x, 0.0).astype(jnp.bfloat16)
    h = _dot(x, w_ref[...])
    grow = (jax.lax.broadcasted_iota(jnp.int32, h.shape, 0)
            + pl.program_id(0) * bm)
    hb = jnp.where(grow < n_real, h, 0.0).astype(jnp.bfloat16)
    h_ref[...] = hb
    al_ref[...] = _dot(hb, av_ref[...])


def _gat_h(x, w_p, av, n_real, np_, bm):
    dp = w_p.shape[1]
    return pl.pallas_call(
        functools.partial(_h_body, n_real, bm),
        grid=(np_ // bm,),
        in_specs=[
            pl.BlockSpec((bm, np_), lambda i: (i, 0)),
            pl.BlockSpec((np_, dp), lambda i: (0, 0)),
            pl.BlockSpec((dp, 128), lambda i: (0, 0)),
        ],
        out_specs=[
            pl.BlockSpec((bm, dp), lambda i: (i, 0)),
            pl.BlockSpec((bm, 128), lambda i: (i, 0)),
        ],
        out_shape=[
            jax.ShapeDtypeStruct((np_, dp), jnp.bfloat16),
            jax.ShapeDtypeStruct((np_, 128), jnp.float32),
        ],
    )(x, w_p, av)


def _gat_attn_body(n_real, adj_ref, alst_ref, al_ref, h_ref, b_ref, o_ref):
    adjt = jnp.transpose(adj_ref[...])
    ad = al_ref[:, 1:2]
    als = alst_ref[0:1, :]
    e = ad + als
    e = jnp.where(e >= 0, e, 0.2 * e)
    e = jnp.where(adjt != 0, e, -1e9)
    col = jax.lax.broadcasted_iota(jnp.int32, e.shape, 1)
    e = jnp.where(col < n_real, e, -2e9)
    p = jnp.exp(e)
    s = jnp.sum(p, axis=1, keepdims=True)
    o = _dot((p / s).astype(jnp.bfloat16), h_ref[...])
    o_ref[...] = jnp.maximum(o + b_ref[0:1, :], 0.0).astype(jnp.bfloat16)


def _gat_attn(adj, alst, al, h_p, brow, n_real, np_, bm):
    dp = h_p.shape[1]
    return pl.pallas_call(
        functools.partial(_gat_attn_body, n_real),
        grid=(np_ // bm,),
        in_specs=[
            pl.BlockSpec((np_, bm), lambda i: (0, i)),
            pl.BlockSpec((8, np_), lambda i: (0, 0)),
            pl.BlockSpec((bm, 128), lambda i: (i, 0)),
            pl.BlockSpec((np_, dp), lambda i: (0, 0)),
            pl.BlockSpec((8, dp), lambda i: (0, 0)),
        ],
        out_specs=pl.BlockSpec((bm, dp), lambda i: (i, 0)),
        out_shape=jax.ShapeDtypeStruct((np_, dp), jnp.bfloat16),
    )(adj, alst, al, h_p, brow)


def _gat(x, adj, w, a_src, a_dst, b, n, np_):
    w_p = _pad2(w, np_, _DP).astype(jnp.bfloat16)
    av = jnp.zeros((_DP, 128), jnp.float32)
    av = av.at[: _D, 0].set(a_src).at[: _D, 1].set(a_dst)
    av = av.astype(jnp.bfloat16)
    h_p, al = _gat_h(x, w_p, av, n, np_, 256)
    alst = jnp.broadcast_to(al[:, 0][None, :], (8, np_))
    brow = _rowvec(b, _DP)
    return _gat_attn(adj, alst, al, h_p, brow, n, np_, 256)


def _qkv_body(x_ref, wqt_ref, wkt_ref, wvt_ref, q_ref, k_ref, v_ref):
    x = x_ref[0]
    q_ref[0] = _dot_t(x, wqt_ref[...]).astype(jnp.bfloat16)
    k_ref[0] = _dot_t(x, wkt_ref[...]).astype(jnp.bfloat16)
    v_ref[0] = _dot_t(x, wvt_ref[...]).astype(jnp.bfloat16)


def _qkv(sum_x_p, wqt, wkt, wvt, bm):
    grid = (_H, _SP // bm)
    io = pl.BlockSpec((1, bm, _DP), lambda h, r: (h, r, 0))
    return pl.pallas_call(
        _qkv_body,
        grid=grid,
        in_specs=[
            pl.BlockSpec((1, bm, _DP), lambda h, r: (h, r, 0)),
            pl.BlockSpec((_DP, _DP), lambda h, r: (0, 0)),
            pl.BlockSpec((_DP, _DP), lambda h, r: (0, 0)),
            pl.BlockSpec((_DP, _DP), lambda h, r: (0, 0)),
        ],
        out_specs=[io, io, io],
        out_shape=[jax.ShapeDtypeStruct((_H, _SP, _DP), jnp.bfloat16)] * 3,
    )(sum_x_p, wqt, wkt, wvt)


def _mha_body(q_ref, k_ref, v_ref, o_ref):
    q = q_ref[0]
    logits = _dot_t(q, k_ref[0]) * (1.0 / math.sqrt(_D))
    col = jax.lax.broadcasted_iota(jnp.int32, logits.shape, 1)
    logits = jnp.where(col < _R, logits, -1e30)
    p = jnp.exp(logits)
    p = p / jnp.sum(p, axis=1, keepdims=True)
    o_ref[0] = _dot(p.astype(jnp.bfloat16), v_ref[0]).astype(jnp.bfloat16)


def _mha(q, k, v, bm):
    grid = (_H, _SP // bm)
    return pl.pallas_call(
        _mha_body,
        grid=grid,
        in_specs=[
            pl.BlockSpec((1, bm, _DP), lambda h, r: (h, r, 0)),
            pl.BlockSpec((1, _SP, _DP), lambda h, r: (h, 0, 0)),
            pl.BlockSpec((1, _SP, _DP), lambda h, r: (h, 0, 0)),
        ],
        out_specs=pl.BlockSpec((1, bm, _DP), lambda h, r: (h, r, 0)),
        out_shape=jax.ShapeDtypeStruct((_H, _SP, _DP), jnp.bfloat16),
    )(q, k, v)


def _wo_body(a_ref, w_ref, b_ref, o_ref, acc_ref):
    hh = pl.program_id(1)

    @pl.when(hh == 0)
    def _():
        acc_ref[...] = jnp.zeros_like(acc_ref)

    acc_ref[...] += _dot_t(a_ref[0], w_ref[0])

    @pl.when(hh == _H - 1)
    def _():
        o_ref[...] = acc_ref[...] + b_ref[0:1, :]


def _wo_proj(attn_h, wo_h, brow, bn):
    return pl.pallas_call(
        _wo_body,
        grid=(_CP // bn, _H),
        in_specs=[
            pl.BlockSpec((1, _SP, _DP), lambda j, hh: (hh, 0, 0)),
            pl.BlockSpec((1, bn, _DP), lambda j, hh: (hh, j, 0)),
            pl.BlockSpec((8, bn), lambda j, hh: (0, j)),
        ],
        out_specs=pl.BlockSpec((_SP, bn), lambda j, hh: (0, j)),
        out_shape=jax.ShapeDtypeStruct((_SP, _CP), jnp.float32),
        scratch_shapes=[pltpu.VMEM((_SP, bn), jnp.float32)],
    )(attn_h, wo_h, brow)


def _mm_body(nk, act, a_ref, wt_ref, b_ref, o_ref, acc_ref):
    @pl.when(pl.program_id(1) == 0)
    def _():
        acc_ref[...] = jnp.zeros_like(acc_ref)

    acc_ref[...] += _dot_t(a_ref[...], wt_ref[...])

    @pl.when(pl.program_id(1) == nk - 1)
    def _():
        o_ref[...] = act(acc_ref[...] + b_ref[0:1, :]).astype(o_ref.dtype)


def _mm_bias(a, wt, brow, act, bn, bk, out_dtype=jnp.float32):
    m, k = a.shape
    n = wt.shape[0]
    nk = k // bk
    return pl.pallas_call(
        functools.partial(_mm_body, nk, act),
        grid=(n // bn, nk),
        in_specs=[
            pl.BlockSpec((m, bk), lambda j, kk: (0, kk)),
            pl.BlockSpec((bn, bk), lambda j, kk: (j, kk)),
            pl.BlockSpec((8, bn), lambda j, kk: (0, j)),
        ],
        out_specs=pl.BlockSpec((m, bn), lambda j, kk: (0, j)),
        out_shape=jax.ShapeDtypeStruct((m, n), out_dtype),
        scratch_shapes=[pltpu.VMEM((m, bn), jnp.float32)],
    )(a, wt, brow)


def _sc_gather(x_p, idx):
    nidx = idx.shape[0]
    cols = x_p.shape[1]
    orig_dtype = x_p.dtype
    if x_p.dtype == jnp.bfloat16:
        x_p = jax.lax.bitcast_convert_type(
            x_p.reshape(x_p.shape[0], cols // 2, 2), jnp.int32)
        cols = cols // 2
    rec = 256 if cols % 256 == 0 else 240
    rpr = cols // rec
    gw = 128
    nrec = nidx * rpr
    x_r = x_p.reshape(-1, rec)
    idx_r = (idx[:, None] * rpr
             + jnp.arange(rpr, dtype=jnp.int32)[None, :]).reshape(1, nrec)
    mesh = plsc.VectorSubcoreMesh(core_axis_name="c", subcore_axis_name="s")

    @functools.partial(
        pl.kernel,
        out_type=jax.ShapeDtypeStruct((nrec, rec), x_p.dtype),
        mesh=mesh,
    )
    def k(x_hbm, i_hbm, o_hbm):
        def body(i_vmem, o_vmem):
            pltpu.sync_copy(x_hbm.at[i_vmem.at[0]], o_vmem)

        pltpu.emit_pipeline(
            body,
            grid=(nrec // gw,),
            in_specs=[pl.BlockSpec((1, gw), index_map=lambda i: (0, i))],
            out_specs=[pl.BlockSpec((gw, rec), index_map=lambda i: (i, 0))],
            core_axis_name=("c", "s"),
            dimension_semantics=(pltpu.PARALLEL,),
        )(i_hbm, o_hbm)

    out = k(x_r, idx_r).reshape(nidx, cols)
    if orig_dtype == jnp.bfloat16:
        out = jax.lax.bitcast_convert_type(
            out, jnp.bfloat16).reshape(nidx, cols * 2)
    return out


def _stats_body(bsz, mirror, g_ref, b_ref, x_ref, alpha_ref, beta_ref,
                xb_ref=None):
    x = x_ref[...].astype(jnp.float32)
    if mirror:
        xb_ref[...] = x.astype(jnp.bfloat16)
    s = jnp.sum(x, axis=0, keepdims=True)
    ss = jnp.sum(x * x, axis=0, keepdims=True)
    mu = s * (1.0 / bsz)
    var = ss * (1.0 / bsz) - mu * mu
    al = g_ref[0:1, :] * jax.lax.rsqrt(var + _EPS)
    be = b_ref[0:1, :] - mu * al
    alpha_ref[...] = jnp.broadcast_to(al, alpha_ref.shape)
    beta_ref[...] = jnp.broadcast_to(be, beta_ref.shape)


def _bn_stats(x, grow, brow, bn, mirror=True):
    bsz, c = x.shape
    outs = [
        pl.BlockSpec((8, bn), lambda j: (0, j)),
        pl.BlockSpec((8, bn), lambda j: (0, j)),
    ]
    shapes = [jax.ShapeDtypeStruct((8, c), jnp.float32)] * 2
    if mirror:
        outs.append(pl.BlockSpec((bsz, bn), lambda j: (0, j)))
        shapes.append(jax.ShapeDtypeStruct((bsz, c), jnp.bfloat16))
    return pl.pallas_call(
        functools.partial(_stats_body, float(bsz), mirror),
        grid=(c // bn,),
        in_specs=[
            pl.BlockSpec((8, bn), lambda j: (0, j)),
            pl.BlockSpec((8, bn), lambda j: (0, j)),
            pl.BlockSpec((bsz, bn), lambda j: (0, j)),
        ],
        out_specs=outs,
        out_shape=shapes,
    )(grow, brow, x)


def _lin1_body(nk, a0_ref, a1_ref, al0_ref, be0_ref, al1_ref, be1_ref,
               w0_ref, w1_ref, bl_ref, o_ref, acc_ref):
    @pl.when(pl.program_id(2) == 0)
    def _():
        acc_ref[...] = jnp.zeros_like(acc_ref)

    z0 = a0_ref[...].astype(jnp.float32) * al0_ref[0:1, :] + be0_ref[0:1, :]
    z1 = a1_ref[...].astype(jnp.float32) * al1_ref[0:1, :] + be1_ref[0:1, :]
    acc_ref[...] += (_dot_t(z0.astype(jnp.bfloat16), w0_ref[...])
                     + _dot_t(z1.astype(jnp.bfloat16), w1_ref[...]))

    @pl.when(pl.program_id(2) == nk - 1)
    def _():
        r = acc_ref[...] + bl_ref[0:1, :]
        o_ref[...] = jnp.where(r >= 0, r, 0.01 * r).astype(jnp.bfloat16)


def _lin1(a0, a1, al0, be0, al1, be1, w0t, w1t, blrow, bm, bn, bk):
    bsz = a0.shape[0]
    kdim = a0.shape[1]
    n = w0t.shape[0]
    nk = kdim // bk
    return pl.pallas_call(
        functools.partial(_lin1_body, nk),
        grid=(bsz // bm, n // bn, nk),
        in_specs=[
            pl.BlockSpec((bm, bk), lambda i, j, kk: (i, kk)),
            pl.BlockSpec((bm, bk), lambda i, j, kk: (i, kk)),
            pl.BlockSpec((8, bk), lambda i, j, kk: (0, kk)),
            pl.BlockSpec((8, bk), lambda i, j, kk: (0, kk)),
            pl.BlockSpec((8, bk), lambda i, j, kk: (0, kk)),
            pl.BlockSpec((8, bk), lambda i, j, kk: (0, kk)),
            pl.BlockSpec((bn, bk), lambda i, j, kk: (j, kk)),
            pl.BlockSpec((bn, bk), lambda i, j, kk: (j, kk)),
            pl.BlockSpec((8, bn), lambda i, j, kk: (0, j)),
        ],
        out_specs=pl.BlockSpec((bm, bn), lambda i, j, kk: (i, j)),
        out_shape=jax.ShapeDtypeStruct((bsz, n), jnp.bfloat16),
        scratch_shapes=[pltpu.VMEM((bm, bn), jnp.float32)],
    )(a0, a1, al0, be0, al1, be1, w0t, w1t, blrow)


def _fin_body(y_ref, al_ref, be_ref, w2_ref, bl2_ref, o_ref):
    z = y_ref[...].astype(jnp.float32) * al_ref[0:1, :] + be_ref[0:1, :]
    r = _dot(z.astype(jnp.bfloat16), w2_ref[...]) + bl2_ref[0:1, :]
    o_ref[...] = jax.nn.sigmoid(r)


def _fin(y, al2, be2, w2col, bl2row, bm):
    bsz, c = y.shape
    return pl.pallas_call(
        _fin_body,
        grid=(bsz // bm,),
        in_specs=[
            pl.BlockSpec((bm, c), lambda i: (i, 0)),
            pl.BlockSpec((8, c), lambda i: (0, 0)),
            pl.BlockSpec((8, c), lambda i: (0, 0)),
            pl.BlockSpec((c, 128), lambda i: (0, 0)),
            pl.BlockSpec((8, 128), lambda i: (0, 0)),
        ],
        out_specs=pl.BlockSpec((bm, 128), lambda i: (i, 0)),
        out_shape=jax.ShapeDtypeStruct((bsz, 128), jnp.float32),
    )(y, al2, be2, w2col, bl2row)


def _mlp_scores(x0, x1, g1, b1, w0t, w1t, blrow, g2row, b2row, w2col,
                bl2row, bm):
    al0, be0 = _bn_stats(x0, g1[0], b1[0], 768, mirror=False)
    al1, be1 = _bn_stats(x1, g1[1], b1[1], 768, mirror=False)
    y = _lin1(x0, x1, al0, be0, al1, be1, w0t, w1t, blrow, bm, 1024, 1280)
    al2, be2 = _bn_stats(y, g2row, b2row, 512, mirror=False)
    return _fin(y, al2, be2, w2col, bl2row, 256)


def kernel(m_drug_d_adj, m_incRNA_d_adj, m_mRNA_d_adj, miRNA_disease_feature,
           x_drug, x_inc, x_mrna,
           Wd, a_src_d, a_dst_d, bd,
           Wi, a_src_i, a_dst_i, bi,
           Wm, a_src_m, a_dst_m, bm_,
           Wq, Wk, Wv, Wo, bo,
           g1, b1, W1, bl1, g2, b2, W2, bl2,
           train_sample, test_sample):
    f32 = jnp.float32
    hd = _gat(x_drug.astype(f32), m_drug_d_adj, Wd, a_src_d, a_dst_d, bd,
              _N1, 2304)
    hi = _gat(x_inc.astype(f32), m_incRNA_d_adj, Wi, a_src_i, a_dst_i, bi,
              _N2, 2560)
    hm = _gat(x_mrna.astype(f32), m_mRNA_d_adj, Wm, a_src_m, a_dst_m, bm_,
              _N3, 4096)

    d_s = jnp.concatenate([hd[:901], hd[1183:_N1]], axis=0)
    i_s = jnp.concatenate([hi[:901], hi[1582:_N2]], axis=0)
    m_s = jnp.concatenate([hm[:901], hm[3052:_N3]], axis=0)
    feat = _pad2(miRNA_disease_feature, _R, _DP).astype(jnp.bfloat16)
    x_heads = jnp.stack([
        jnp.pad(d_s, ((0, _SP - _R), (0, 0))),
        jnp.pad(i_s, ((0, _SP - _R), (0, 0))),
        jnp.pad(m_s, ((0, _SP - _R), (0, 0))),
        jnp.pad(feat, ((0, _SP - _R), (0, 0))),
    ])

    wqt = _pad2(Wq, _DP, _DP).astype(jnp.bfloat16)
    wkt = _pad2(Wk, _DP, _DP).astype(jnp.bfloat16)
    wvt = _pad2(Wv, _DP, _DP).astype(jnp.bfloat16)
    q, k, v = _qkv(x_heads, wqt, wkt, wvt, 256)
    attn = _mha(q, k, v, 256)

    wo_h = jnp.stack([
        _pad2(Wo[:, hh * _D:(hh + 1) * _D], _CP, _DP) for hh in range(_H)
    ]).astype(jnp.bfloat16)
    sum_x = _wo_proj(attn, wo_h, _rowvec(bo, _CP), 768)

    ts = train_sample.astype(jnp.int32)
    us = test_sample.astype(jnp.int32)
    tr0 = _sc_gather(sum_x, ts[:, 0])
    tr1 = _sc_gather(sum_x, ts[:, 1])
    te0 = _sc_gather(sum_x, us[:, 0])
    te1 = _sc_gather(sum_x, us[:, 1])

    g1h = (_rowvec(g1[:_EMB], _CP), _rowvec(g1[_EMB:], _CP))
    b1h = (_rowvec(b1[:_EMB], _CP), _rowvec(b1[_EMB:], _CP))
    w0t = _pad2(W1[:, :_EMB], _EMBP, _CP).astype(jnp.bfloat16)
    w1t = _pad2(W1[:, _EMB:], _EMBP, _CP).astype(jnp.bfloat16)
    blrow = _rowvec(bl1, _EMBP)
    g2row = _rowvec(g2, _EMBP)
    b2row = _rowvec(b2, _EMBP)
    w2col = jnp.zeros((_EMBP, 128), f32).at[:_EMB, 0].set(W2[0]).astype(jnp.bfloat16)
    bl2row = jnp.broadcast_to(bl2[0], (8, 128)).astype(f32)

    tr_score = _mlp_scores(tr0, tr1, g1h, b1h, w0t, w1t, blrow,
                           g2row, b2row, w2col, bl2row, 1024)
    te_score = _mlp_scores(te0, te1, g1h, b1h, w0t, w1t, blrow,
                           g2row, b2row, w2col, bl2row, 1024)
    return tr_score[:, :1], te_score[:, :1]
